# Initial kernel scaffold; baseline (speedup 1.0000x reference)
import jax
import jax.numpy as jnp
from jax.experimental import pallas as pl


def kernel(input_ids, token_table, position_table):
    raise NotImplementedError("write your pallas kernel here")


# speedup vs baseline: 2.2477x; 2.2477x over previous
import functools

import jax
import jax.numpy as jnp
from jax import lax
from jax.experimental import pallas as pl
from jax.experimental.pallas import tpu as pltpu
from jax.experimental.pallas import tpu_sc as plsc

HIDDEN = 128
MAX_POS = 4096
BATCH = 4
SEQ = 4096

NC, NS, L = 2, 16, 16
NW = NC * NS
TOK = BATCH * SEQ
ROWS_PER_W = TOK // NW
CHUNK = 128
NCHUNK = ROWS_PER_W // CHUNK


def _sc_embed(ids2d, token_table, position_table):
    mesh = plsc.VectorSubcoreMesh(core_axis_name="c", subcore_axis_name="s")

    @functools.partial(
        pl.kernel,
        mesh=mesh,
        out_type=jax.ShapeDtypeStruct((TOK, HIDDEN), jnp.float32),
        scratch_types=[
            pltpu.VMEM((NCHUNK, CHUNK), jnp.int32),
            pltpu.VMEM((CHUNK, HIDDEN), jnp.float32),
            pltpu.VMEM((CHUNK, HIDDEN), jnp.float32),
            pltpu.SemaphoreType.DMA,
        ],
    )
    def body(ids_hbm, tok_hbm, pos_hbm, out_hbm, idx_v, tok_v, acc_v, sem):
        wid = lax.axis_index("s") * NC + lax.axis_index("c")
        base = wid * ROWS_PER_W
        pos_base = lax.rem(base, MAX_POS)

        pltpu.sync_copy(ids_hbm.at[pl.ds(wid * NCHUNK, NCHUNK)], idx_v)
        for j in range(NCHUNK):
            gat = pltpu.async_copy(tok_hbm.at[idx_v.at[j]], tok_v, sem)
            pltpu.sync_copy(pos_hbm.at[pl.ds(pos_base + j * CHUNK, CHUNK)],
                            acc_v)
            gat.wait()

            def add_row(r, carry):
                for c in range(HIDDEN // L):
                    sl = (r, pl.ds(c * L, L))
                    acc_v[sl] = acc_v[sl] + tok_v[sl]
                return carry

            lax.fori_loop(0, CHUNK, add_row, 0)
            pltpu.sync_copy(acc_v, out_hbm.at[pl.ds(base + j * CHUNK, CHUNK)])

    return body(ids2d, token_table, position_table)


def kernel(input_ids, token_table, position_table):
    ids2d = input_ids.astype(jnp.int32).reshape(TOK // CHUNK, CHUNK)
    out = _sc_embed(ids2d, token_table, position_table)
    return out.reshape(BATCH, SEQ, HIDDEN)


# speedup vs baseline: 2.4621x; 1.0954x over previous
import functools

import jax
import jax.numpy as jnp
from jax import lax
from jax.experimental import pallas as pl
from jax.experimental.pallas import tpu as pltpu
from jax.experimental.pallas import tpu_sc as plsc

HIDDEN = 128
MAX_POS = 4096
BATCH = 4
SEQ = 4096

NC, NS, L = 2, 16, 16
NW = NC * NS
TOK = BATCH * SEQ
ROWS_PER_W = TOK // NW
CHUNK = 128
NCHUNK = ROWS_PER_W // CHUNK
NACC = 3


def _sc_embed(ids2d, token_table, position_table):
    mesh = plsc.VectorSubcoreMesh(core_axis_name="c", subcore_axis_name="s")

    @functools.partial(
        pl.kernel,
        mesh=mesh,
        out_type=jax.ShapeDtypeStruct((TOK, HIDDEN), jnp.float32),
        scratch_types=(
            [pltpu.VMEM((NCHUNK, CHUNK), jnp.int32)]
            + [pltpu.VMEM((CHUNK, HIDDEN), jnp.float32)] * NCHUNK
            + [pltpu.VMEM((CHUNK, HIDDEN), jnp.float32)] * NACC
            + [pltpu.SemaphoreType.DMA] * (NCHUNK + NACC + NACC)
        ),
    )
    def body(ids_hbm, tok_hbm, pos_hbm, out_hbm, idx_v, *scratch):
        tok_v = scratch[:NCHUNK]
        acc_v = scratch[NCHUNK:NCHUNK + NACC]
        gsem = scratch[NCHUNK + NACC:2 * NCHUNK + NACC]
        psem = scratch[2 * NCHUNK + NACC:2 * NCHUNK + 2 * NACC]
        ssem = scratch[2 * NCHUNK + 2 * NACC:]

        wid = lax.axis_index("s") * NC + lax.axis_index("c")
        base = wid * ROWS_PER_W
        pos_base = lax.rem(base, MAX_POS)

        pltpu.sync_copy(ids_hbm.at[pl.ds(wid * NCHUNK, NCHUNK)], idx_v)
        gats = [
            pltpu.async_copy(tok_hbm.at[idx_v.at[j]], tok_v[j], gsem[j])
            for j in range(NCHUNK)
        ]
        poss = {}
        for j in range(NACC):
            poss[j] = pltpu.async_copy(
                pos_hbm.at[pl.ds(pos_base + j * CHUNK, CHUNK)],
                acc_v[j], psem[j])
        stores = {}
        for j in range(NCHUNK):
            if j >= NACC:
                stores[j - NACC].wait()
                poss[j] = pltpu.async_copy(
                    pos_hbm.at[pl.ds(pos_base + j * CHUNK, CHUNK)],
                    acc_v[j % NACC], psem[j % NACC])
            poss[j].wait()
            gats[j].wait()
            a, t = acc_v[j % NACC], tok_v[j]

            def add_row(r, carry, a=a, t=t):
                for c in range(HIDDEN // L):
                    sl = (r, pl.ds(c * L, L))
                    a[sl] = a[sl] + t[sl]
                return carry

            lax.fori_loop(0, CHUNK, add_row, 0)
            stores[j] = pltpu.async_copy(
                a, out_hbm.at[pl.ds(base + j * CHUNK, CHUNK)],
                ssem[j % NACC])
        for j in range(max(0, NCHUNK - NACC), NCHUNK):
            stores[j].wait()

    return body(ids2d, token_table, position_table)


def kernel(input_ids, token_table, position_table):
    ids2d = input_ids.astype(jnp.int32).reshape(TOK // CHUNK, CHUNK)
    out = _sc_embed(ids2d, token_table, position_table)
    return out.reshape(BATCH, SEQ, HIDDEN)
